# baseline (speedup 1.0000x reference)
import functools

import jax
import jax.numpy as jnp
from jax import lax
from jax.experimental import pallas as pl
from jax.experimental.pallas import tpu as pltpu
from jax.experimental.pallas import tpu_sc as plsc

VOCAB = 1000000
DIM = 64
B = 4096
L = 200
N = B * L

_info = plsc.get_sparse_core_info()
NC, NS = _info.num_cores, _info.num_subcores
NW = NC * NS
N_PER_W = N // NW
CHUNK = 128
STEPS = N_PER_W // CHUNK


def _make_kernel():
    mesh = plsc.VectorSubcoreMesh(core_axis_name="c", subcore_axis_name="s")

    @functools.partial(
        pl.kernel,
        mesh=mesh,
        out_type=jax.ShapeDtypeStruct((N, DIM), jnp.float32),
        compiler_params=pltpu.CompilerParams(use_tc_tiling_on_sc=False),
        scratch_types=[
            pltpu.VMEM((CHUNK,), jnp.int32),
            pltpu.VMEM((CHUNK, DIM), jnp.float32),
            pltpu.SemaphoreType.DMA,
        ],
    )
    def emb_kernel(idx_hbm, table_hbm, out_hbm, idx_v, rows_v, sem):
        wid = lax.axis_index("s") * NC + lax.axis_index("c")
        base = wid * N_PER_W

        def body(g, carry):
            start = base + g * CHUNK
            pltpu.sync_copy(idx_hbm.at[pl.ds(start, CHUNK)], idx_v)
            pltpu.async_copy(table_hbm.at[idx_v], rows_v, sem).wait()
            pltpu.sync_copy(rows_v, out_hbm.at[pl.ds(start, CHUNK)])
            return carry

        lax.fori_loop(0, STEPS, body, 0)

    return emb_kernel


_emb_kernel = _make_kernel()


def kernel(x_input, table):
    idx = x_input.reshape(N).astype(jnp.int32)
    out = _emb_kernel(idx, table)
    return out.reshape(B, L * DIM)


# speedup vs baseline: 1.2455x; 1.2455x over previous
import functools

import jax
import jax.numpy as jnp
from jax import lax
from jax.experimental import pallas as pl
from jax.experimental.pallas import tpu as pltpu
from jax.experimental.pallas import tpu_sc as plsc

VOCAB = 1000000
DIM = 64
B = 4096
L = 200
N = B * L

_info = plsc.get_sparse_core_info()
NC, NS = _info.num_cores, _info.num_subcores
NW = NC * NS
N_PER_W = N // NW
CHUNK = 128
STEPS = N_PER_W // CHUNK
NBUF = 4
ROUNDS = STEPS // NBUF


def _make_kernel():
    mesh = plsc.VectorSubcoreMesh(core_axis_name="c", subcore_axis_name="s")

    @functools.partial(
        pl.kernel,
        mesh=mesh,
        out_type=jax.ShapeDtypeStruct((N, DIM), jnp.float32),
        compiler_params=pltpu.CompilerParams(use_tc_tiling_on_sc=False),
        scratch_types=[
            pltpu.VMEM((STEPS, CHUNK), jnp.int32),
            pltpu.VMEM((NBUF, CHUNK, DIM), jnp.float32),
        ]
        + [pltpu.SemaphoreType.DMA] * NBUF
        + [pltpu.SemaphoreType.DMA] * NBUF,
    )
    def emb_kernel(idx_hbm, table_hbm, out_hbm, idx_v, rows_v, *sems):
        sem_g = sems[:NBUF]
        sem_o = sems[NBUF:]
        wid = lax.axis_index("s") * NC + lax.axis_index("c")
        base = wid * N_PER_W

        pltpu.sync_copy(idx_hbm.at[wid], idx_v)

        def gather_start(j, b):
            pltpu.make_async_copy(
                table_hbm.at[idx_v.at[j]], rows_v.at[b], sem_g[b]
            ).start()

        def gather_wait(j, b):
            pltpu.make_async_copy(
                table_hbm.at[idx_v.at[j]], rows_v.at[b], sem_g[b]
            ).wait()

        def out_start(j, b):
            pltpu.make_async_copy(
                rows_v.at[b], out_hbm.at[pl.ds(base + j * CHUNK, CHUNK)], sem_o[b]
            ).start()

        def out_wait(j, b):
            pltpu.make_async_copy(
                rows_v.at[b], out_hbm.at[pl.ds(base + j * CHUNK, CHUNK)], sem_o[b]
            ).wait()

        for b in range(NBUF):
            gather_start(b, b)

        def body(r, carry):
            j0 = r * NBUF
            for b in range(NBUF):
                gather_wait(j0 + b, b)
                out_start(j0 + b, b)
            for b in range(NBUF):
                out_wait(j0 + b, b)
                gather_start(j0 + NBUF + b, b)
            return carry

        lax.fori_loop(0, ROUNDS - 1, body, 0)

        j0 = (ROUNDS - 1) * NBUF
        for b in range(NBUF):
            gather_wait(j0 + b, b)
            out_start(j0 + b, b)
        for b in range(NBUF):
            out_wait(j0 + b, b)

    return emb_kernel


_emb_kernel = _make_kernel()


def kernel(x_input, table):
    idx = x_input.reshape(NW, STEPS, CHUNK).astype(jnp.int32)
    out = _emb_kernel(idx, table)
    return out.reshape(B, L * DIM)


# speedup vs baseline: 1.7102x; 1.3731x over previous
import functools

import jax
import jax.numpy as jnp
from jax import lax
from jax.experimental import pallas as pl
from jax.experimental.pallas import tpu as pltpu
from jax.experimental.pallas import tpu_sc as plsc

VOCAB = 1000000
DIM = 64
B = 4096
L = 200
N = B * L

WBLK = 4096
HBLK = WBLK // 2
NBLK = (VOCAB + WBLK - 1) // WBLK
VPAD = NBLK * WBLK

_info = plsc.get_sparse_core_info()
NC, NS = _info.num_cores, _info.num_subcores
NW = NC * NS
N_PER_W = N // NW
CHUNK = 128
STEPS = N_PER_W // CHUNK
NBUF = 4
ROUNDS = STEPS // NBUF


def _transpose_body(tt_ref, out_ref):
    a = tt_ref[:, :HBLK]
    b = tt_ref[:, HBLK:]
    out_ref[:, :DIM] = a.T
    out_ref[:, DIM:] = b.T


_pack_table = pl.pallas_call(
    _transpose_body,
    grid=(NBLK,),
    in_specs=[pl.BlockSpec((DIM, WBLK), lambda q: (0, q))],
    out_specs=pl.BlockSpec((HBLK, 2 * DIM), lambda q: (q, 0)),
    out_shape=jax.ShapeDtypeStruct((VPAD // 2, 2 * DIM), jnp.float32),
)


def _make_gather_kernel():
    mesh = plsc.VectorSubcoreMesh(core_axis_name="c", subcore_axis_name="s")

    @functools.partial(
        pl.kernel,
        mesh=mesh,
        out_type=jax.ShapeDtypeStruct((N, DIM), jnp.float32),
        compiler_params=pltpu.CompilerParams(use_tc_tiling_on_sc=False),
        scratch_types=[
            pltpu.VMEM((STEPS, CHUNK), jnp.int32),
            pltpu.VMEM((NBUF, CHUNK, DIM), jnp.float32),
        ]
        + [pltpu.SemaphoreType.DMA] * NBUF
        + [pltpu.SemaphoreType.DMA] * NBUF,
    )
    def emb_kernel(idx_hbm, table_hbm, out_hbm, idx_v, rows_v, *sems):
        sem_g = sems[:NBUF]
        sem_o = sems[NBUF:]
        wid = lax.axis_index("s") * NC + lax.axis_index("c")
        base = wid * N_PER_W

        pltpu.sync_copy(idx_hbm.at[wid], idx_v)

        def gather_start(j, b):
            pltpu.make_async_copy(
                table_hbm.at[idx_v.at[j]], rows_v.at[b], sem_g[b]
            ).start()

        def gather_wait(j, b):
            pltpu.make_async_copy(
                table_hbm.at[idx_v.at[j]], rows_v.at[b], sem_g[b]
            ).wait()

        def out_start(j, b):
            pltpu.make_async_copy(
                rows_v.at[b], out_hbm.at[pl.ds(base + j * CHUNK, CHUNK)], sem_o[b]
            ).start()

        def out_wait(j, b):
            pltpu.make_async_copy(
                rows_v.at[b], out_hbm.at[pl.ds(base + j * CHUNK, CHUNK)], sem_o[b]
            ).wait()

        for b in range(NBUF):
            gather_start(b, b)

        def body(r, carry):
            j0 = r * NBUF
            for b in range(NBUF):
                gather_wait(j0 + b, b)
                out_start(j0 + b, b)
            for b in range(NBUF):
                out_wait(j0 + b, b)
                gather_start(j0 + NBUF + b, b)
            return carry

        lax.fori_loop(0, ROUNDS - 1, body, 0)

        j0 = (ROUNDS - 1) * NBUF
        for b in range(NBUF):
            gather_wait(j0 + b, b)
            out_start(j0 + b, b)
        for b in range(NBUF):
            out_wait(j0 + b, b)

    return emb_kernel


_emb_gather = _make_gather_kernel()


def kernel(x_input, table):
    packed = _pack_table(table.T)
    table_lin = packed.reshape(VPAD, DIM)
    v = x_input.reshape(N).astype(jnp.int32)
    f = (v & ~(WBLK - 1)) | ((v & (HBLK - 1)) << 1) | ((v >> 11) & 1)
    idx = f.reshape(NW, STEPS, CHUNK)
    out = _emb_gather(idx, table_lin)
    return out.reshape(B, L * DIM)


# speedup vs baseline: 1.8598x; 1.0875x over previous
import functools

import jax
import jax.numpy as jnp
from jax import lax
from jax.experimental import pallas as pl
from jax.experimental.pallas import tpu as pltpu
from jax.experimental.pallas import tpu_sc as plsc

VOCAB = 1000000
DIM = 64
B = 4096
L = 200
N = B * L

WBLK = 4096
HBLK = WBLK // 2
NBLK = (VOCAB + WBLK - 1) // WBLK
VPAD = NBLK * WBLK

_info = plsc.get_sparse_core_info()
NC, NS = _info.num_cores, _info.num_subcores
NW = NC * NS
N_PER_W = N // NW
CHUNK = 128
STEPS = N_PER_W // CHUNK
NBUF = 4
ROUNDS = STEPS // NBUF


def _transpose_body(tt_ref, out_ref):
    a = tt_ref[:, :HBLK]
    b = tt_ref[:, HBLK:]
    c = jnp.concatenate([a, b], axis=0)
    out_ref[...] = c.T


_pack_table = pl.pallas_call(
    _transpose_body,
    grid=(NBLK,),
    in_specs=[pl.BlockSpec((DIM, WBLK), lambda q: (0, q))],
    out_specs=pl.BlockSpec((HBLK, 2 * DIM), lambda q: (q, 0)),
    out_shape=jax.ShapeDtypeStruct((VPAD // 2, 2 * DIM), jnp.float32),
)


def _make_gather_kernel():
    mesh = plsc.VectorSubcoreMesh(core_axis_name="c", subcore_axis_name="s")

    @functools.partial(
        pl.kernel,
        mesh=mesh,
        out_type=jax.ShapeDtypeStruct((N, DIM), jnp.float32),
        compiler_params=pltpu.CompilerParams(use_tc_tiling_on_sc=False),
        scratch_types=[
            pltpu.VMEM((STEPS, CHUNK), jnp.int32),
            pltpu.VMEM((NBUF, CHUNK, DIM), jnp.float32),
        ]
        + [pltpu.SemaphoreType.DMA] * NBUF
        + [pltpu.SemaphoreType.DMA] * NBUF,
    )
    def emb_kernel(idx_hbm, table_hbm, out_hbm, idx_v, rows_v, *sems):
        sem_g = sems[:NBUF]
        sem_o = sems[NBUF:]
        wid = lax.axis_index("s") * NC + lax.axis_index("c")
        base = wid * N_PER_W

        pltpu.sync_copy(idx_hbm.at[wid], idx_v)

        def gather_start(j, b):
            pltpu.make_async_copy(
                table_hbm.at[idx_v.at[j]], rows_v.at[b], sem_g[b]
            ).start()

        def gather_wait(j, b):
            pltpu.make_async_copy(
                table_hbm.at[idx_v.at[j]], rows_v.at[b], sem_g[b]
            ).wait()

        def out_start(j, b):
            pltpu.make_async_copy(
                rows_v.at[b], out_hbm.at[pl.ds(base + j * CHUNK, CHUNK)], sem_o[b]
            ).start()

        def out_wait(j, b):
            pltpu.make_async_copy(
                rows_v.at[b], out_hbm.at[pl.ds(base + j * CHUNK, CHUNK)], sem_o[b]
            ).wait()

        for b in range(NBUF):
            gather_start(b, b)

        def body(r, carry):
            j0 = r * NBUF
            for b in range(NBUF):
                gather_wait(j0 + b, b)
                out_start(j0 + b, b)
            for b in range(NBUF):
                out_wait(j0 + b, b)
                gather_start(j0 + NBUF + b, b)
            return carry

        lax.fori_loop(0, ROUNDS - 1, body, 0)

        j0 = (ROUNDS - 1) * NBUF
        for b in range(NBUF):
            gather_wait(j0 + b, b)
            out_start(j0 + b, b)
        for b in range(NBUF):
            out_wait(j0 + b, b)

    return emb_kernel


_emb_gather = _make_gather_kernel()


def kernel(x_input, table):
    packed = _pack_table(table.T)
    table_lin = packed.reshape(VPAD, DIM)
    v = x_input.reshape(N).astype(jnp.int32)
    f = (v & ~(WBLK - 1)) | ((v & (HBLK - 1)) << 1) | ((v >> 11) & 1)
    idx = f.reshape(NW, STEPS, CHUNK)
    out = _emb_gather(idx, table_lin)
    return out.reshape(B, L * DIM)


# speedup vs baseline: 2.2134x; 1.1902x over previous
import functools

import jax
import jax.numpy as jnp
from jax import lax
from jax.experimental import pallas as pl
from jax.experimental.pallas import tpu as pltpu
from jax.experimental.pallas import tpu_sc as plsc

VOCAB = 1000000
DIM = 64
B = 4096
L = 200
N = B * L

WBLK = 16384
HBLK = WBLK // 2
NBLK = (VOCAB + WBLK - 1) // WBLK
VPAD = NBLK * WBLK

_info = plsc.get_sparse_core_info()
NC, NS = _info.num_cores, _info.num_subcores
NW = NC * NS
N_PER_W = N // NW
CHUNK = 128
STEPS = N_PER_W // CHUNK
NBUF = 4
ROUNDS = STEPS // NBUF


def _transpose_body(tt_ref, out_ref):
    a = tt_ref[:, :HBLK]
    b = tt_ref[:, HBLK:]
    c = jnp.concatenate([a, b], axis=0)
    out_ref[...] = c.T


_pack_table = pl.pallas_call(
    _transpose_body,
    grid=(NBLK,),
    in_specs=[pl.BlockSpec((DIM, WBLK), lambda q: (0, q))],
    out_specs=pl.BlockSpec((HBLK, 2 * DIM), lambda q: (q, 0)),
    out_shape=jax.ShapeDtypeStruct((VPAD // 2, 2 * DIM), jnp.float32),
)


def _make_gather_kernel():
    mesh = plsc.VectorSubcoreMesh(core_axis_name="c", subcore_axis_name="s")

    @functools.partial(
        pl.kernel,
        mesh=mesh,
        out_type=jax.ShapeDtypeStruct((N, DIM), jnp.float32),
        compiler_params=pltpu.CompilerParams(use_tc_tiling_on_sc=False),
        scratch_types=[
            pltpu.VMEM((STEPS, CHUNK), jnp.int32),
            pltpu.VMEM((NBUF, CHUNK, DIM), jnp.float32),
        ]
        + [pltpu.SemaphoreType.DMA] * NBUF
        + [pltpu.SemaphoreType.DMA] * NBUF,
    )
    def emb_kernel(idx_hbm, table_hbm, out_hbm, idx_v, rows_v, *sems):
        sem_g = sems[:NBUF]
        sem_o = sems[NBUF:]
        wid = lax.axis_index("s") * NC + lax.axis_index("c")
        base = wid * N_PER_W

        pltpu.sync_copy(idx_hbm.at[wid], idx_v)

        def gather_start(j, b):
            pltpu.make_async_copy(
                table_hbm.at[idx_v.at[j]], rows_v.at[b], sem_g[b]
            ).start()

        def gather_wait(j, b):
            pltpu.make_async_copy(
                table_hbm.at[idx_v.at[j]], rows_v.at[b], sem_g[b]
            ).wait()

        def out_start(j, b):
            pltpu.make_async_copy(
                rows_v.at[b], out_hbm.at[pl.ds(base + j * CHUNK, CHUNK)], sem_o[b]
            ).start()

        def out_wait(j, b):
            pltpu.make_async_copy(
                rows_v.at[b], out_hbm.at[pl.ds(base + j * CHUNK, CHUNK)], sem_o[b]
            ).wait()

        for b in range(NBUF):
            gather_start(b, b)

        def body(r, carry):
            j0 = r * NBUF
            for b in range(NBUF):
                gather_wait(j0 + b, b)
                out_start(j0 + b, b)
            for b in range(NBUF):
                out_wait(j0 + b, b)
                gather_start(j0 + NBUF + b, b)
            return carry

        lax.fori_loop(0, ROUNDS - 1, body, 0)

        j0 = (ROUNDS - 1) * NBUF
        for b in range(NBUF):
            gather_wait(j0 + b, b)
            out_start(j0 + b, b)
        for b in range(NBUF):
            out_wait(j0 + b, b)

    return emb_kernel


_emb_gather = _make_gather_kernel()


def kernel(x_input, table):
    packed = _pack_table(table.T)
    table_lin = packed.reshape(VPAD, DIM)
    v = x_input.reshape(N).astype(jnp.int32)
    f = (v & ~(WBLK - 1)) | ((v & (HBLK - 1)) << 1) | ((v // HBLK) & 1)
    idx = f.reshape(NW, STEPS, CHUNK)
    out = _emb_gather(idx, table_lin)
    return out.reshape(B, L * DIM)
